# baseline (speedup 1.0000x reference)
import jax
import jax.numpy as jnp
from jax.experimental import pallas as pl
from jax.experimental.pallas import tpu as pltpu
from jax.experimental.pallas import tpu_sc as plsc

_NE = 8192
_D = 256
_N = 8192
_BR = 256
_NB = _N // _BR
_LOSS_SCALE = 1.25 / (_N * _D)
_ST = 256

_SC_INFO = plsc.get_sparse_core_info()
_NW = _SC_INFO.num_cores * _SC_INFO.num_subcores
_BPW = _N // _NW


def _vq_body(x_ref, w_ref, enc_ref, idx_ref, loss_ref, w2_ref, acc_ref):
    i = pl.program_id(0)

    @pl.when(i == 0)
    def _init():
        w = w_ref[...]
        w2_ref[...] = jnp.sum(w * w, axis=1, keepdims=True).T
        acc_ref[...] = jnp.zeros_like(acc_ref)

    xb = x_ref[...]
    x2 = jnp.sum(xb * xb, axis=1, keepdims=True)
    xb2 = xb * -2.0
    w2r = w2_ref[...]
    dn = (((1,), (1,)), ((), ()))

    def _strip(c):
        mm2 = jax.lax.dot_general(xb2, w_ref[pl.ds(c * _ST, _ST), :], dn)
        return (x2 + w2r[:, c * _ST:(c + 1) * _ST]) + mm2

    run_min = _strip(0)
    run_c = jnp.zeros((_BR, _ST), jnp.int32)
    for c in range(1, _NE // _ST):
        d = _strip(c)
        m = d < run_min
        run_min = jnp.where(m, d, run_min)
        run_c = jnp.where(m, c, run_c)
    lane = jax.lax.broadcasted_iota(jnp.int32, (_BR, _ST), 1)
    cand = run_c * _ST + lane
    dmin = jnp.min(run_min, axis=1, keepdims=True)
    idx = jnp.min(jnp.where(run_min == dmin, cand, _NE), axis=1, keepdims=True)
    idx_ref[...] = idx
    col = jax.lax.broadcasted_iota(jnp.int32, (_BR, _NE), 1)
    enc_ref[...] = (col == idx).astype(jnp.float32)
    acc_ref[...] += jnp.sum(dmin, axis=0, keepdims=True)

    @pl.when(i == _NB - 1)
    def _fin():
        loss_ref[...] = acc_ref[...] * _LOSS_SCALE


_HPW = _BPW // 2


def _gather_body(w_hbm, idx_hbm, q_hbm, idx0, idx1, rows0, rows1, sem0, sem1):
    wid = jax.lax.axis_index("s") * _SC_INFO.num_cores + jax.lax.axis_index("c")
    base = wid * _BPW
    pltpu.sync_copy(idx_hbm.at[pl.ds(base, _HPW)], idx0)
    pltpu.sync_copy(idx_hbm.at[pl.ds(base + _HPW, _HPW)], idx1)
    c0 = pltpu.async_copy(w_hbm.at[idx0], rows0, sem0)
    c1 = pltpu.async_copy(w_hbm.at[idx1], rows1, sem1)
    c0.wait()
    pltpu.sync_copy(rows0, q_hbm.at[pl.ds(base, _HPW)])
    c1.wait()
    pltpu.sync_copy(rows1, q_hbm.at[pl.ds(base + _HPW, _HPW)])


_sc_gather = pl.kernel(
    _gather_body,
    out_type=jax.ShapeDtypeStruct((_N, _D), jnp.float32),
    mesh=plsc.VectorSubcoreMesh(core_axis_name="c", subcore_axis_name="s"),
    scratch_types=[
        pltpu.VMEM((_HPW,), jnp.int32),
        pltpu.VMEM((_HPW,), jnp.int32),
        pltpu.VMEM((_HPW, _D), jnp.float32),
        pltpu.VMEM((_HPW, _D), jnp.float32),
        pltpu.SemaphoreType.DMA,
        pltpu.SemaphoreType.DMA,
    ],
)


def kernel(x, W):
    flat_x = jnp.transpose(x, (0, 2, 3, 1)).reshape(_N, _D)
    enc, idx, loss = pl.pallas_call(
        _vq_body,
        grid=(_NB,),
        in_specs=[
            pl.BlockSpec((_BR, _D), lambda i: (i, 0)),
            pl.BlockSpec((_NE, _D), lambda i: (0, 0)),
        ],
        out_specs=[
            pl.BlockSpec((_BR, _NE), lambda i: (i, 0)),
            pl.BlockSpec((_BR, 1), lambda i: (i, 0)),
            pl.BlockSpec((1, 1), lambda i: (0, 0)),
        ],
        out_shape=[
            jax.ShapeDtypeStruct((_N, _NE), jnp.float32),
            jax.ShapeDtypeStruct((_N, 1), jnp.int32),
            jax.ShapeDtypeStruct((1, 1), jnp.float32),
        ],
        scratch_shapes=[pltpu.VMEM((1, _NE), jnp.float32),
                        pltpu.VMEM((1, 1), jnp.float32)],
    )(flat_x, W)
    q = _sc_gather(W, idx.reshape(_N))
    quantized = jnp.transpose(q.reshape(8, 32, 32, _D), (0, 3, 1, 2))
    return (loss[0, 0], quantized, enc)


# speedup vs baseline: 1.0530x; 1.0530x over previous
import jax
import jax.numpy as jnp
from jax.experimental import pallas as pl
from jax.experimental.pallas import tpu as pltpu
from jax.experimental.pallas import tpu_sc as plsc

_NE = 8192
_D = 256
_N = 8192
_BR = 512
_NB = _N // _BR
_LOSS_SCALE = 1.25 / (_N * _D)
_ST = 256

_SC_INFO = plsc.get_sparse_core_info()
_NW = _SC_INFO.num_cores * _SC_INFO.num_subcores
_BPW = _N // _NW


def _vq_body(x_ref, w_ref, enc_ref, idx_ref, loss_ref, w2_ref, acc_ref):
    i = pl.program_id(0)

    @pl.when(i == 0)
    def _init():
        w = w_ref[...]
        w2_ref[...] = jnp.sum(w * w, axis=1, keepdims=True).T
        acc_ref[...] = jnp.zeros_like(acc_ref)

    xb = x_ref[...]
    x2 = jnp.sum(xb * xb, axis=1, keepdims=True)
    xb2 = xb * -2.0
    w2r = w2_ref[...]
    dn = (((1,), (1,)), ((), ()))

    def _strip(c):
        mm2 = jax.lax.dot_general(xb2, w_ref[pl.ds(c * _ST, _ST), :], dn)
        return (x2 + w2r[:, c * _ST:(c + 1) * _ST]) + mm2

    run_min = _strip(0)
    run_c = jnp.zeros((_BR, _ST), jnp.int32)
    for c in range(1, _NE // _ST):
        d = _strip(c)
        m = d < run_min
        run_min = jnp.where(m, d, run_min)
        run_c = jnp.where(m, c, run_c)
    lane = jax.lax.broadcasted_iota(jnp.int32, (_BR, _ST), 1)
    cand = run_c * _ST + lane
    dmin = jnp.min(run_min, axis=1, keepdims=True)
    idx = jnp.min(jnp.where(run_min == dmin, cand, _NE), axis=1, keepdims=True)
    idx_ref[...] = idx
    col = jax.lax.broadcasted_iota(jnp.int32, (_BR, _NE), 1)
    enc_ref[...] = (col == idx).astype(jnp.float32)
    acc_ref[...] += jnp.sum(dmin, axis=0, keepdims=True)

    @pl.when(i == _NB - 1)
    def _fin():
        loss_ref[...] = acc_ref[...] * _LOSS_SCALE


def _gather_body(w_hbm, idx_hbm, q_hbm, idx_v, rows_v, sem):
    wid = jax.lax.axis_index("s") * _SC_INFO.num_cores + jax.lax.axis_index("c")
    base = wid * _BPW
    pltpu.sync_copy(idx_hbm.at[pl.ds(base, _BPW)], idx_v)
    pltpu.async_copy(w_hbm.at[idx_v], rows_v, sem).wait()
    pltpu.sync_copy(rows_v, q_hbm.at[pl.ds(base, _BPW)])


_sc_gather = pl.kernel(
    _gather_body,
    out_type=jax.ShapeDtypeStruct((_N, _D), jnp.float32),
    mesh=plsc.VectorSubcoreMesh(core_axis_name="c", subcore_axis_name="s"),
    scratch_types=[
        pltpu.VMEM((_BPW,), jnp.int32),
        pltpu.VMEM((_BPW, _D), jnp.float32),
        pltpu.SemaphoreType.DMA,
    ],
)


def kernel(x, W):
    flat_x = jnp.transpose(x, (0, 2, 3, 1)).reshape(_N, _D)
    enc, idx, loss = pl.pallas_call(
        _vq_body,
        grid=(_NB,),
        in_specs=[
            pl.BlockSpec((_BR, _D), lambda i: (i, 0)),
            pl.BlockSpec((_NE, _D), lambda i: (0, 0)),
        ],
        out_specs=[
            pl.BlockSpec((_BR, _NE), lambda i: (i, 0)),
            pl.BlockSpec((_BR, 1), lambda i: (i, 0)),
            pl.BlockSpec((1, 1), lambda i: (0, 0)),
        ],
        out_shape=[
            jax.ShapeDtypeStruct((_N, _NE), jnp.float32),
            jax.ShapeDtypeStruct((_N, 1), jnp.int32),
            jax.ShapeDtypeStruct((1, 1), jnp.float32),
        ],
        scratch_shapes=[pltpu.VMEM((1, _NE), jnp.float32),
                        pltpu.VMEM((1, 1), jnp.float32)],
    )(flat_x, W)
    q = _sc_gather(W, idx.reshape(_N))
    quantized = jnp.transpose(q.reshape(8, 32, 32, _D), (0, 3, 1, 2))
    return (loss[0, 0], quantized, enc)
